# baseline (speedup 1.0000x reference)
import functools

import jax
import jax.numpy as jnp
from jax import lax
from jax.experimental import pallas as pl
from jax.experimental.pallas import tpu as pltpu
from jax.experimental.pallas import tpu_sc as plsc

_HIDDEN = 768
_E = 8
_NT = 32768
_BT = 2048
_GRID = _NT // _BT

_NSUB = 16
_CHUNK = (_NT * 2) // _NSUB
_UNROLL = 4
_NGRP = _CHUNK // (16 * _UNROLL)


def _dense_body(x_ref, w_ref, idx_ref, wts_ref, isc_ref, avg_ref, ent_ref):
    i = pl.program_id(0)
    x = x_ref[...]
    w = w_ref[...]

    lt = jax.lax.dot_general(
        w, x, (((1,), (1,)), ((), ())),
        preferred_element_type=jnp.float32)

    m = jnp.max(lt, axis=0, keepdims=True)
    ex = jnp.exp(lt - m)
    z = jnp.sum(ex, axis=0, keepdims=True)
    p = ex / z

    m1 = p[0:1, :]
    e1 = jnp.zeros((1, _BT), jnp.int32)
    for e in range(1, _E):
        gt = p[e:e + 1, :] > m1
        m1 = jnp.where(gt, p[e:e + 1, :], m1)
        e1 = jnp.where(gt, e, e1)
    m2 = jnp.full((1, _BT), -1.0, jnp.float32)
    e2 = jnp.zeros((1, _BT), jnp.int32)
    for e in range(_E):
        cand = jnp.where(e1 == e, -1.0, p[e:e + 1, :])
        gt = cand > m2
        m2 = jnp.where(gt, cand, m2)
        e2 = jnp.where(gt, e, e2)

    s = m1 + m2
    widx = jnp.concatenate([e1, e2], axis=0)
    wwts = jnp.concatenate([m1 / s, m2 / s], axis=0)
    isc_ref[...] = widx
    idx_ref[...] = widx.T
    wts_ref[...] = wwts.T

    p_part = jnp.sum(p, axis=1, keepdims=True)
    ent_part = -jnp.sum(p * jnp.log(p + 1e-10)).reshape(1, 1)

    @pl.when(i == 0)
    def _init():
        avg_ref[...] = p_part
        ent_ref[...] = ent_part

    @pl.when(i > 0)
    def _acc():
        avg_ref[...] += p_part
        ent_ref[...] += ent_part

    @pl.when(i == _GRID - 1)
    def _final():
        avg_ref[...] = avg_ref[...] / _NT
        ent_ref[...] = ent_ref[...] / _NT


def _dense(x, w):
    out_shapes = (
        jax.ShapeDtypeStruct((_NT, 2), jnp.int32),
        jax.ShapeDtypeStruct((_NT, 2), jnp.float32),
        jax.ShapeDtypeStruct((2, _NT), jnp.int32),
        jax.ShapeDtypeStruct((_E, 1), jnp.float32),
        jax.ShapeDtypeStruct((1, 1), jnp.float32),
    )
    out_specs = (
        pl.BlockSpec((_BT, 2), lambda i: (i, 0)),
        pl.BlockSpec((_BT, 2), lambda i: (i, 0)),
        pl.BlockSpec((2, _BT), lambda i: (0, i)),
        pl.BlockSpec((_E, 1), lambda i: (0, 0)),
        pl.BlockSpec((1, 1), lambda i: (0, 0)),
    )
    in_specs = [
        pl.BlockSpec((_BT, _HIDDEN), lambda i: (i, 0)),
        pl.BlockSpec((_E, _HIDDEN), lambda i: (0, 0)),
    ]
    return pl.pallas_call(
        _dense_body,
        grid=(_GRID,),
        in_specs=in_specs,
        out_specs=out_specs,
        out_shape=out_shapes,
    )(x, w)


def _count_body(idx_hbm, cnt_hbm, gini_hbm, idx_vm, loc_vm, shared_vm, acc_vm):
    wid = lax.axis_index("s")
    row = wid // (_NSUB // 2)
    off = (wid % (_NSUB // 2)) * _CHUNK

    pltpu.sync_copy(idx_hbm.at[row, pl.ds(off, _CHUNK)], idx_vm)

    lane = lax.iota(jnp.int32, 16)
    zero16 = jnp.zeros((16,), jnp.float32)

    def group(g, cnts):
        out = list(cnts)
        for u in range(_UNROLL):
            v = idx_vm[pl.ds((g * _UNROLL + u) * 16, 16)]
            for e in range(_E):
                out[e] = out[e] + jnp.where(v == e, 1.0, 0.0)
        return tuple(out)

    cnts = lax.fori_loop(0, _NGRP, group, tuple(zero16 for _ in range(_E)))

    local = zero16
    for e in range(_E):
        tot = jnp.sum(cnts[e])
        local = jnp.where(lane == e, tot, local)
    loc_vm[...] = local

    pltpu.sync_copy(loc_vm, shared_vm.at[pl.ds(wid * 16, 16)])
    plsc.subcore_barrier()

    @pl.when(wid == 0)
    def _finish():
        pltpu.sync_copy(shared_vm, acc_vm)
        counts = acc_vm[pl.ds(0, 16)]
        for r in range(1, _NSUB):
            counts = counts + acc_vm[pl.ds(r * 16, 16)]
        total = jnp.sum(counts)

        keys = jnp.where(lane < _E, counts, 1e30)
        skeys, _ = plsc.sort_key_val(keys, lane)
        lane_f = lane.astype(jnp.float32)
        coef = jnp.where(lane < _E, 2.0 * (lane_f + 1.0) - _E - 1.0, 0.0)
        num_v = jnp.full((16,), jnp.sum(coef * skeys), jnp.float32)
        den_v = jnp.full((16,), _E * total + 1e-10, jnp.float32)
        gini_v = num_v / den_v

        loc_vm[...] = counts
        pltpu.sync_copy(loc_vm, cnt_hbm)
        loc_vm[...] = gini_v
        pltpu.sync_copy(loc_vm, gini_hbm)


@functools.partial(
    pl.kernel,
    mesh=plsc.VectorSubcoreMesh(
        core_axis_name="c", subcore_axis_name="s", num_cores=1),
    compiler_params=pltpu.CompilerParams(
        needs_layout_passes=False, skip_device_barrier=True),
    out_type=(
        jax.ShapeDtypeStruct((16,), jnp.float32),
        jax.ShapeDtypeStruct((16,), jnp.float32),
    ),
    scratch_types=[
        pltpu.VMEM((_CHUNK,), jnp.int32),
        pltpu.VMEM((16,), jnp.float32),
        pltpu.VMEM_SHARED((_NSUB * 16,), jnp.float32),
        pltpu.VMEM((_NSUB * 16,), jnp.float32),
    ],
)
def _count(idx_hbm, cnt_hbm, gini_hbm, idx_vm, loc_vm, shared_vm, acc_vm):
    _count_body(idx_hbm, cnt_hbm, gini_hbm, idx_vm, loc_vm, shared_vm, acc_vm)


@jax.jit
def _router(x, w):
    idx, wts, idx_sc, avg, ent = _dense(x, w)
    cnt, gini = _count(idx_sc)
    return (idx, wts, cnt[:_E], avg.reshape(_E),
            ent.reshape(()), gini[0].reshape(()))


def kernel(x, W):
    return _router(x, W)


# speedup vs baseline: 1.4866x; 1.4866x over previous
import functools

import jax
import jax.numpy as jnp
from jax import lax
from jax.experimental import pallas as pl
from jax.experimental.pallas import tpu as pltpu
from jax.experimental.pallas import tpu_sc as plsc

_HIDDEN = 768
_E = 8
_NT = 32768
_BT = 4096
_GRID = _NT // _BT

_NSUB = 16
_CHUNK = (_NT * 2) // _NSUB
_UNROLL = 4
_NGRP = _CHUNK // (16 * _UNROLL)


def _dense_body(x_ref, w_ref, isc_ref, wsc_ref, avg_ref, ent_ref):
    i = pl.program_id(0)
    x = x_ref[...]
    w = w_ref[...]

    lt = jax.lax.dot_general(
        w, x, (((1,), (1,)), ((), ())),
        preferred_element_type=jnp.float32)

    m = jnp.max(lt, axis=0, keepdims=True)
    ex = jnp.exp(lt - m)
    z = jnp.sum(ex, axis=0, keepdims=True)
    p = ex / z

    m1 = p[0:1, :]
    e1 = jnp.zeros((1, _BT), jnp.int32)
    for e in range(1, _E):
        gt = p[e:e + 1, :] > m1
        m1 = jnp.where(gt, p[e:e + 1, :], m1)
        e1 = jnp.where(gt, e, e1)
    m2 = jnp.full((1, _BT), -1.0, jnp.float32)
    e2 = jnp.zeros((1, _BT), jnp.int32)
    for e in range(_E):
        cand = jnp.where(e1 == e, -1.0, p[e:e + 1, :])
        gt = cand > m2
        m2 = jnp.where(gt, cand, m2)
        e2 = jnp.where(gt, e, e2)

    s = m1 + m2
    isc_ref[...] = jnp.concatenate([e1, e2], axis=0)
    wsc_ref[...] = jnp.concatenate([m1 / s, m2 / s], axis=0)

    p_part = jnp.sum(p, axis=1, keepdims=True)
    ent_part = -jnp.sum(p * jnp.log(p + 1e-10)).reshape(1, 1)

    @pl.when(i == 0)
    def _init():
        avg_ref[...] = p_part
        ent_ref[...] = ent_part

    @pl.when(i > 0)
    def _acc():
        avg_ref[...] += p_part
        ent_ref[...] += ent_part

    @pl.when(i == _GRID - 1)
    def _final():
        avg_ref[...] = avg_ref[...] / _NT
        ent_ref[...] = ent_ref[...] / _NT


def _dense(x, w):
    out_shapes = (
        jax.ShapeDtypeStruct((2, _NT), jnp.int32),
        jax.ShapeDtypeStruct((2, _NT), jnp.float32),
        jax.ShapeDtypeStruct((_E, 1), jnp.float32),
        jax.ShapeDtypeStruct((1, 1), jnp.float32),
    )
    out_specs = (
        pl.BlockSpec((2, _BT), lambda i: (0, i)),
        pl.BlockSpec((2, _BT), lambda i: (0, i)),
        pl.BlockSpec((_E, 1), lambda i: (0, 0)),
        pl.BlockSpec((1, 1), lambda i: (0, 0)),
    )
    in_specs = [
        pl.BlockSpec((_BT, _HIDDEN), lambda i: (i, 0)),
        pl.BlockSpec((_E, _HIDDEN), lambda i: (0, 0)),
    ]
    return pl.pallas_call(
        _dense_body,
        grid=(_GRID,),
        in_specs=in_specs,
        out_specs=out_specs,
        out_shape=out_shapes,
    )(x, w)


def _count_body(idx_hbm, cnt_hbm, gini_hbm, idx_vm, loc_vm, shared_vm, acc_vm):
    wid = lax.axis_index("s")
    row = wid // (_NSUB // 2)
    off = (wid % (_NSUB // 2)) * _CHUNK

    pltpu.sync_copy(idx_hbm.at[row, pl.ds(off, _CHUNK)], idx_vm)

    lane = lax.iota(jnp.int32, 16)
    zero16 = jnp.zeros((16,), jnp.float32)

    def group(g, cnts):
        out = list(cnts)
        for u in range(_UNROLL):
            v = idx_vm[pl.ds((g * _UNROLL + u) * 16, 16)]
            for e in range(_E):
                out[e] = out[e] + jnp.where(v == e, 1.0, 0.0)
        return tuple(out)

    cnts = lax.fori_loop(0, _NGRP, group, tuple(zero16 for _ in range(_E)))

    local = zero16
    for e in range(_E):
        tot = jnp.sum(cnts[e])
        local = jnp.where(lane == e, tot, local)
    loc_vm[...] = local

    pltpu.sync_copy(loc_vm, shared_vm.at[pl.ds(wid * 16, 16)])
    plsc.subcore_barrier()

    @pl.when(wid == 0)
    def _finish():
        pltpu.sync_copy(shared_vm, acc_vm)
        counts = acc_vm[pl.ds(0, 16)]
        for r in range(1, _NSUB):
            counts = counts + acc_vm[pl.ds(r * 16, 16)]
        total = jnp.sum(counts)

        keys = jnp.where(lane < _E, counts, 1e30)
        skeys, _ = plsc.sort_key_val(keys, lane)
        lane_f = lane.astype(jnp.float32)
        coef = jnp.where(lane < _E, 2.0 * (lane_f + 1.0) - _E - 1.0, 0.0)
        num_v = jnp.full((16,), jnp.sum(coef * skeys), jnp.float32)
        den_v = jnp.full((16,), _E * total + 1e-10, jnp.float32)
        gini_v = num_v / den_v

        loc_vm[...] = counts
        pltpu.sync_copy(loc_vm, cnt_hbm)
        loc_vm[...] = gini_v
        pltpu.sync_copy(loc_vm, gini_hbm)


@functools.partial(
    pl.kernel,
    mesh=plsc.VectorSubcoreMesh(
        core_axis_name="c", subcore_axis_name="s", num_cores=1),
    compiler_params=pltpu.CompilerParams(needs_layout_passes=False),
    out_type=(
        jax.ShapeDtypeStruct((16,), jnp.float32),
        jax.ShapeDtypeStruct((16,), jnp.float32),
    ),
    scratch_types=[
        pltpu.VMEM((_CHUNK,), jnp.int32),
        pltpu.VMEM((16,), jnp.float32),
        pltpu.VMEM_SHARED((_NSUB * 16,), jnp.float32),
        pltpu.VMEM((_NSUB * 16,), jnp.float32),
    ],
)
def _count(idx_hbm, cnt_hbm, gini_hbm, idx_vm, loc_vm, shared_vm, acc_vm):
    _count_body(idx_hbm, cnt_hbm, gini_hbm, idx_vm, loc_vm, shared_vm, acc_vm)


@jax.jit
def _router(x, w):
    idx_sc, wts_sc, avg, ent = _dense(x, w)
    cnt, gini = _count(idx_sc)
    return (idx_sc.T, wts_sc.T, cnt[:_E], avg.reshape(_E),
            ent.reshape(()), gini[0].reshape(()))


def kernel(x, W):
    return _router(x, W)
